# Initial kernel scaffold; baseline (speedup 1.0000x reference)
import jax
import jax.numpy as jnp
from jax.experimental import pallas as pl


def kernel(x, lt, all_nodes):
    raise NotImplementedError("write your pallas kernel here")


# speedup vs baseline: 1.2599x; 1.2599x over previous
import functools

import jax
import jax.numpy as jnp
from jax import lax
from jax.experimental import pallas as pl
from jax.experimental.pallas import tpu as pltpu
from jax.experimental.pallas import tpu_sc as plsc

N_NODES = 50000
DIM = 128
D_FEAT = 256
D_OUT = DIM + D_FEAT

CHUNK = 128
NUM_FULL = N_NODES // CHUNK
TAIL = N_NODES - NUM_FULL * CHUNK
NUM_CHUNKS = NUM_FULL + (1 if TAIL else 0)
TAIL_BASE = N_NODES - CHUNK


@functools.lru_cache(maxsize=None)
def _build():
    mesh = plsc.VectorSubcoreMesh(core_axis_name="c", subcore_axis_name="s")
    nc, ns = mesh.num_cores, mesh.num_subcores
    nw = nc * ns
    iters = -(-NUM_CHUNKS // nw)

    @functools.partial(
        pl.kernel,
        out_type=jax.ShapeDtypeStruct((N_NODES, D_OUT), jnp.float32),
        mesh=mesh,
        scratch_types=[
            pltpu.VMEM((CHUNK,), jnp.int32),
            pltpu.VMEM((CHUNK, DIM), jnp.float32),
            pltpu.VMEM((CHUNK, D_FEAT), jnp.float32),
            pltpu.SemaphoreType.DMA,
        ],
    )
    def body(x_hbm, lt_hbm, idx_hbm, out_hbm, idx_v, h_v, x_v, sem):
        wid = lax.axis_index("s") * nc + lax.axis_index("c")
        for i in range(iters):
            chunk = wid + i * nw

            @pl.when(chunk < NUM_CHUNKS)
            def _():
                base = jnp.where(chunk < NUM_FULL, chunk * CHUNK, TAIL_BASE)
                base = pl.multiple_of(base, 8)
                pltpu.sync_copy(idx_hbm.at[pl.ds(base, CHUNK)], idx_v)
                pltpu.async_copy(lt_hbm.at[idx_v], h_v, sem).wait()
                pltpu.sync_copy(h_v, out_hbm.at[pl.ds(base, CHUNK), pl.ds(0, DIM)])
                pltpu.sync_copy(x_hbm.at[pl.ds(base, CHUNK)], x_v)
                pltpu.sync_copy(x_v, out_hbm.at[pl.ds(base, CHUNK), pl.ds(DIM, D_FEAT)])

    return body


def kernel(x, lt, all_nodes):
    idx32 = all_nodes.astype(jnp.int32)
    return _build()(x, lt, idx32)
